# baseline (speedup 1.0000x reference)
import functools

import jax
import jax.numpy as jnp
from jax.experimental import pallas as pl
from jax.experimental.pallas import tpu as pltpu

NZ, NY, NX = 10, 100, 100
NVOX = NZ * NY * NX
NVOXP = 100352
XR0, XR1 = -40.0, 40.0
YR0, YR1 = -40.0, 40.0
ZR0, ZR1 = -2.0, 6.0
VS = 0.8

N = 33600
NPAD = 33792
CHUNK = 2400
NCHUNKS = N // CHUNK
CB = 128
NCB = 256 // CB
TILE = 2048


def _lerp4(v00, v01, v10, v11):
    top = v00 * 0.5 + v01 * 0.5
    bot = v10 * 0.5 + v11 * 0.5
    return top * 0.5 + bot * 0.5


def _prep_body(inp_ref, lin_ref, w_ref):
    g = inp_ref[...]
    x = _lerp4(g[0], g[1], g[2], g[3])
    y = _lerp4(g[4], g[5], g[6], g[7])
    z = _lerp4(g[8], g[9], g[10], g[11])
    cf = _lerp4(g[12], g[13], g[14], g[15])
    valid = jnp.isfinite(x) & jnp.isfinite(y) & jnp.isfinite(z)
    valid = valid & (cf > 1e-4)
    valid = valid & (x >= XR0) & (x < XR1)
    valid = valid & (y >= YR0) & (y < YR1)
    valid = valid & (z >= ZR0) & (z < ZR1)
    ix = jnp.clip(jnp.floor((x - XR0) / VS).astype(jnp.int32), 0, NX - 1)
    iy = jnp.clip(jnp.floor((y - YR0) / VS).astype(jnp.int32), 0, NY - 1)
    iz = jnp.clip(jnp.floor((z - ZR0) / VS).astype(jnp.int32), 0, NZ - 1)
    lin = iz * (NY * NX) + iy * NX + ix
    lin_ref[...] = jnp.where(valid, lin, 0)
    w_ref[...] = cf * valid.astype(jnp.float32)


def _scatter_body(lin_ref, w_ref, feat_ref, vol_ref, wsum_ref, acc, sem):
    cb = pl.program_id(0)
    ch = pl.program_id(1)

    @pl.when(ch == 0)
    def _():
        acc[...] = jnp.zeros((NVOXP, CB), jnp.float32)

    base = ch * CHUNK

    def body_feat(i, carry):
        ln = lin_ref[base + i]
        wt = w_ref[base + i]
        acc[pl.ds(ln, 1), :] += feat_ref[pl.ds(i, 1), :] * wt
        return carry

    def body_wsum(i, carry):
        ln = lin_ref[base + i]
        wt = w_ref[base + i]
        acc[pl.ds(ln, 1), :] += jnp.full((1, CB), wt, jnp.float32)
        return carry

    @pl.when(cb < NCB)
    def _():
        jax.lax.fori_loop(0, CHUNK, body_feat, 0)

    @pl.when(cb == NCB)
    def _():
        jax.lax.fori_loop(0, CHUNK, body_wsum, 0)

    @pl.when((ch == NCHUNKS - 1) & (cb < NCB))
    def _():
        cp = pltpu.make_async_copy(acc, vol_ref.at[:, pl.ds(cb * CB, CB)], sem)
        cp.start()
        cp.wait()

    @pl.when((ch == NCHUNKS - 1) & (cb == NCB))
    def _():
        cp = pltpu.make_async_copy(acc, wsum_ref, sem)
        cp.start()
        cp.wait()


def _proj_body(vol_ref, wsum_ref, pw_ref, pb_ref, out_ref):
    wcol = wsum_ref[:, 0:1]
    voln = vol_ref[...] / jnp.maximum(wcol, 1e-6)
    mm = jax.lax.dot_general(
        pw_ref[...], voln,
        dimension_numbers=(((1,), (1,)), ((), ())),
        preferred_element_type=jnp.float32,
    )
    out_ref[...] = mm + pb_ref[...]


@jax.jit
def _lift(inp, feat_flat, proj_w, proj_b):
    lin2, w2 = pl.pallas_call(
        _prep_body,
        out_shape=[
            jax.ShapeDtypeStruct((NPAD // 128, 128), jnp.int32),
            jax.ShapeDtypeStruct((NPAD // 128, 128), jnp.float32),
        ],
    )(inp)
    lin_s = lin2.reshape(-1)[:N]
    w_s = w2.reshape(-1)[:N]

    vol, wsum = pl.pallas_call(
        _scatter_body,
        grid=(NCB + 1, NCHUNKS),
        in_specs=[
            pl.BlockSpec((N,), lambda cb, ch: (0,),
                         memory_space=pltpu.SMEM),
            pl.BlockSpec((N,), lambda cb, ch: (0,),
                         memory_space=pltpu.SMEM),
            pl.BlockSpec((CHUNK, CB), lambda cb, ch: (ch, cb % NCB)),
        ],
        out_specs=[
            pl.BlockSpec(memory_space=pl.ANY),
            pl.BlockSpec(memory_space=pl.ANY),
        ],
        out_shape=[
            jax.ShapeDtypeStruct((NVOXP, 256), jnp.float32),
            jax.ShapeDtypeStruct((NVOXP, CB), jnp.float32),
        ],
        scratch_shapes=[
            pltpu.VMEM((NVOXP, CB), jnp.float32),
            pltpu.SemaphoreType.DMA,
        ],
    )(lin_s, w_s, feat_flat)

    out = pl.pallas_call(
        _proj_body,
        grid=(NVOXP // TILE,),
        in_specs=[
            pl.BlockSpec((TILE, 256), lambda i: (i, 0)),
            pl.BlockSpec((TILE, CB), lambda i: (i, 0)),
            pl.BlockSpec((256, 256), lambda i: (0, 0)),
            pl.BlockSpec((256, 1), lambda i: (0, 0)),
        ],
        out_specs=pl.BlockSpec((256, TILE), lambda i: (0, i)),
        out_shape=jax.ShapeDtypeStruct((256, NVOXP), jnp.float32),
    )(vol, wsum, proj_w, proj_b.reshape(256, 1))
    return out[:, :NVOX]


def kernel(feat_1_4, points, points_conf, proj_w, proj_b):
    b, t, v, c, h4, w4 = feat_1_4.shape
    h, w = points.shape[3], points.shape[4]
    f32 = jnp.float32

    P = points.reshape(v, h, w, 3).astype(f32)
    Cf = points_conf.reshape(v, h, w).astype(f32)

    rows = []
    taps = [(1, 1), (1, 2), (2, 1), (2, 2)]
    comps = [[], [], []]
    for (r, s) in taps:
        tp = P[:, r::4, s::4, :]
        st = tp.transpose(0, 2, 3, 1).reshape(-1, 3)
        for k in range(3):
            comps[k].append(st[:, k])
    for k in range(3):
        rows.extend(comps[k])
    for (r, s) in taps:
        rows.append(Cf[:, r::4, s::4].reshape(-1))
    inp = jnp.stack(rows)
    inp = jnp.pad(inp, ((0, 0), (0, NPAD - N)))
    inp = inp.reshape(16, NPAD // 128, 128)

    feat_flat = (feat_1_4.reshape(v, c, h4, w4)
                 .transpose(0, 2, 3, 1).reshape(N, c).astype(f32))

    out = _lift(inp, feat_flat, proj_w.astype(f32), proj_b.astype(f32))
    return out.reshape(1, 1, c, NZ, NY, NX).astype(feat_1_4.dtype)
